# speedup vs baseline: 16.1420x; 16.1420x over previous
import jax
import jax.numpy as jnp
from jax.experimental import pallas as pl

N = 20000
MAX_OUT = 100
IOU_THRESH = 0.7

ROWS = 160
LANES = 128
NPAD = ROWS * LANES


def _nms_body(x1_ref, y1_ref, x2_ref, y2_ref, sc_ref, out_ref):
    x1 = x1_ref[...]
    y1 = y1_ref[...]
    x2 = x2_ref[...]
    y2 = y2_ref[...]
    sc = sc_ref[...]

    valid = (x2 > x1 + 1.0) & (y2 > y1 + 1.0)
    neg_inf = jnp.float32(-jnp.inf)
    work0 = jnp.where(valid, sc, neg_inf)
    areas = jnp.clip(x2 - x1, 0.0) * jnp.clip(y2 - y1, 0.0)

    ridx = jax.lax.broadcasted_iota(jnp.int32, (ROWS, LANES), 0)
    cidx = jax.lax.broadcasted_iota(jnp.int32, (ROWS, LANES), 1)
    idx = ridx * LANES + cidx
    lane = jax.lax.broadcasted_iota(jnp.int32, (1, LANES), 1)

    def body(i, work):
        m = jnp.max(work)
        cand = jnp.where(work == m, idx, jnp.int32(2**30))
        bi = jnp.min(cand)
        wm = idx == bi
        bx1 = jnp.sum(jnp.where(wm, x1, 0.0))
        by1 = jnp.sum(jnp.where(wm, y1, 0.0))
        bx2 = jnp.sum(jnp.where(wm, x2, 0.0))
        by2 = jnp.sum(jnp.where(wm, y2, 0.0))
        bar = jnp.sum(jnp.where(wm, areas, 0.0))
        bsc = jnp.sum(jnp.where(wm, sc, 0.0))

        ix1 = jnp.maximum(x1, bx1)
        iy1 = jnp.maximum(y1, by1)
        ix2 = jnp.minimum(x2, bx2)
        iy2 = jnp.minimum(y2, by2)
        inter = jnp.clip(ix2 - ix1, 0.0) * jnp.clip(iy2 - iy1, 0.0)
        iou = inter / (areas + bar - inter + 1e-8)
        work = jnp.where(iou > IOU_THRESH, neg_inf, work)
        work = jnp.where(wm, neg_inf, work)

        row = jnp.zeros((1, LANES), jnp.float32)
        for k, v in enumerate([bx1, by1, bx2, by2, bsc]):
            row = jnp.where(lane == k, v, row)
        out_ref[pl.ds(i, 1), :] = row
        return work

    jax.lax.fori_loop(0, MAX_OUT, body, work0)


def kernel(boxes, scores):
    pad = NPAD - N
    x1 = jnp.pad(boxes[:, 0], (0, pad)).reshape(ROWS, LANES)
    y1 = jnp.pad(boxes[:, 1], (0, pad)).reshape(ROWS, LANES)
    x2 = jnp.pad(boxes[:, 2], (0, pad)).reshape(ROWS, LANES)
    y2 = jnp.pad(boxes[:, 3], (0, pad)).reshape(ROWS, LANES)
    sc = jnp.pad(scores, (0, pad)).reshape(ROWS, LANES)

    out = pl.pallas_call(
        _nms_body,
        out_shape=jax.ShapeDtypeStruct((MAX_OUT, LANES), jnp.float32),
    )(x1, y1, x2, y2, sc)
    return out[:, :5]
